# Initial kernel scaffold; baseline (speedup 1.0000x reference)
import jax
import jax.numpy as jnp
from jax.experimental import pallas as pl


def kernel(x, edge_index, pos_edge, neg_edge, Wl1, bl1, Wr1, Wl2, bl2, Wr2, Wd1, bd1, Wd2, bd2):
    raise NotImplementedError("write your pallas kernel here")


# speedup vs baseline: 3.6481x; 3.6481x over previous
import functools

import jax
import jax.numpy as jnp
from jax import lax
from jax.experimental import pallas as pl
from jax.experimental.pallas import tpu as pltpu
from jax.experimental.pallas import tpu_sc as plsc

N = 10000
E = 320000
D = 128
NC = 2
NS = 16
NW = NC * NS
CH = 80
RB = 1024
NP = 10240
RPT = NP // NS

_mesh = plsc.VectorSubcoreMesh(
    core_axis_name="c", subcore_axis_name="s", num_cores=NC, num_subcores=NS)


def _enc_prep_body(x_ref, wl_ref, wr_ref, bl_ref, y_ref, r_ref):
    xb = x_ref[...]
    y_ref[...] = jnp.dot(xb, wl_ref[...].T, preferred_element_type=jnp.float32)
    r_ref[...] = (jnp.dot(xb, wr_ref[...].T, preferred_element_type=jnp.float32)
                  + bl_ref[...])


def _inv_count_col(cnt_blk):
    ones = jnp.ones((NW, 1), jnp.float32)
    col = lax.dot_general(cnt_blk, ones, (((0,), (0,)), ((), ())),
                          preferred_element_type=jnp.float32)
    return 1.0 / jnp.maximum(col, 1.0)


def _combine1_body(p_ref, cnt_ref, r1_ref, wl2_ref, wr2_ref, bl2_ref,
                   y2_ref, r2_ref, invb_ref):
    inv = _inv_count_col(cnt_ref[...])
    agg = p_ref[0] + p_ref[1]
    h = jnp.maximum(agg * inv + r1_ref[...], 0.0)
    y2_ref[...] = jnp.dot(h, wl2_ref[...].T, preferred_element_type=jnp.float32)
    r2_ref[...] = (jnp.dot(h, wr2_ref[...].T, preferred_element_type=jnp.float32)
                   + bl2_ref[...])
    invb_ref[...] = jnp.broadcast_to(inv, (inv.shape[0], D))


def _combine2_body(p_ref, r2_ref, invb_ref, w1a_ref, w1b_ref, bd1_ref,
                   a_ref, b_ref):
    agg = p_ref[0] + p_ref[1]
    z = agg * invb_ref[...] + r2_ref[...]
    a_ref[...] = jnp.dot(z, w1a_ref[...].T, preferred_element_type=jnp.float32)
    b_ref[...] = (jnp.dot(z, w1b_ref[...].T, preferred_element_type=jnp.float32)
                  + bd1_ref[...])


def _row_spec(width):
    return pl.BlockSpec((RB, width), lambda i: (i, 0))


def _full_spec(shape):
    return pl.BlockSpec(shape, lambda i: tuple(0 for _ in shape))


_enc_prep = pl.pallas_call(
    _enc_prep_body,
    grid=(pl.cdiv(N, RB),),
    in_specs=[_row_spec(D), _full_spec((D, D)), _full_spec((D, D)),
              _full_spec((1, D))],
    out_specs=[_row_spec(D), _row_spec(D)],
    out_shape=[jax.ShapeDtypeStruct((N, D), jnp.float32),
               jax.ShapeDtypeStruct((N, D), jnp.float32)],
)

_combine1 = pl.pallas_call(
    _combine1_body,
    grid=(pl.cdiv(N, RB),),
    in_specs=[pl.BlockSpec((NC, RB, D), lambda i: (0, i, 0)),
              pl.BlockSpec((NW, RB), lambda i: (0, i)),
              _row_spec(D), _full_spec((D, D)), _full_spec((D, D)),
              _full_spec((1, D))],
    out_specs=[_row_spec(D), _row_spec(D), _row_spec(D)],
    out_shape=[jax.ShapeDtypeStruct((N, D), jnp.float32),
               jax.ShapeDtypeStruct((N, D), jnp.float32),
               jax.ShapeDtypeStruct((N, D), jnp.float32)],
)

_combine2 = pl.pallas_call(
    _combine2_body,
    grid=(pl.cdiv(N, RB),),
    in_specs=[pl.BlockSpec((NC, RB, D), lambda i: (0, i, 0)),
              _row_spec(D), _row_spec(D), _full_spec((D, D)),
              _full_spec((D, D)), _full_spec((1, D))],
    out_specs=[_row_spec(D), _row_spec(D)],
    out_shape=[jax.ShapeDtypeStruct((N, D), jnp.float32),
               jax.ShapeDtypeStruct((N, D), jnp.float32)],
)


def _zero_spmem_slice(acc_sh, zrow_v, s):
    zr = zrow_v.shape[0]

    def zb(i, carry):
        for j in range(D // 16):
            zrow_v[i, pl.ds(j * 16, 16)] = jnp.zeros((16,), jnp.float32)
        return carry

    lax.fori_loop(0, zr, zb, 0)

    def zcp(k, carry):
        off = pl.multiple_of(s * RPT + k * zr, 8)
        pltpu.sync_copy(zrow_v, acc_sh.at[pl.ds(off, zr)])
        return carry

    lax.fori_loop(0, RPT // zr, zcp, 0)


def _make_segsum(with_count):
    epw = E // NW

    def body(y_hbm, src_hbm, dst_hbm, *refs):
        if with_count:
            (agg_hbm, cnt_hbm,
             acc_sh, zrow_v, idx_s, idx_d, rows_v, hist_v, sem) = refs
        else:
            (agg_hbm, acc_sh, zrow_v, idx_s, idx_d, rows_v, sem) = refs
        c = lax.axis_index("c")
        s = lax.axis_index("s")
        wid = c * NS + s
        _zero_spmem_slice(acc_sh, zrow_v, s)
        if with_count:
            def zh(i, carry):
                hist_v[pl.ds(i * 16, 16)] = jnp.zeros((16,), jnp.float32)
                return carry

            lax.fori_loop(0, NP // 16, zh, 0)
        plsc.subcore_barrier()

        def chunk(k, carry):
            base = pl.multiple_of(wid * epw + k * CH, 8)
            pltpu.sync_copy(src_hbm.at[pl.ds(base, CH)], idx_s)
            pltpu.sync_copy(dst_hbm.at[pl.ds(base, CH)], idx_d)
            gat = pltpu.async_copy(y_hbm.at[idx_s], rows_v, sem)
            if with_count:
                for t in range(CH // 16):
                    d16 = idx_d[pl.ds(t * 16, 16)]
                    cnts, lastm = plsc.scan_count(d16)
                    plsc.addupdate_scatter(hist_v, [d16],
                                           cnts.astype(jnp.float32),
                                           mask=lastm)
            gat.wait()
            pltpu.sync_copy(rows_v, acc_sh.at[idx_d], add=True)
            return carry

        lax.fori_loop(0, epw // CH, chunk, 0)
        if with_count:
            pltpu.sync_copy(hist_v, cnt_hbm.at[wid])
        plsc.subcore_barrier()
        off = pl.multiple_of(s * RPT, 8)
        pltpu.sync_copy(acc_sh.at[pl.ds(off, RPT)],
                        agg_hbm.at[c, pl.ds(off, RPT)])

    out_type = jax.ShapeDtypeStruct((NC, NP, D), jnp.float32)
    scratch = [
        pltpu.VMEM_SHARED((NP, D), jnp.float32),
        pltpu.VMEM((128, D), jnp.float32),
        pltpu.VMEM((CH,), jnp.int32),
        pltpu.VMEM((CH,), jnp.int32),
        pltpu.VMEM((CH, D), jnp.float32),
    ]
    if with_count:
        out_type = [out_type, jax.ShapeDtypeStruct((NW, NP), jnp.float32)]
        scratch = scratch + [pltpu.VMEM((NP,), jnp.float32)]
    scratch = scratch + [pltpu.SemaphoreType.DMA]

    return pl.kernel(
        body, out_type=out_type, mesh=_mesh, scratch_types=scratch,
        compiler_params=pltpu.CompilerParams(needs_layout_passes=False))


_segsum_cnt = _make_segsum(True)
_segsum = _make_segsum(False)


def _decoder_body(a_hbm, b_hbm, u_hbm, v_hbm, w2_hbm, b2_hbm, out_hbm,
                  w2_v, b2_v, idx_u, idx_v, arows, brows, outb, sem):
    ne = 2 * E
    dpw = ne // NW
    c = lax.axis_index("c")
    s = lax.axis_index("s")
    wid = c * NS + s
    pltpu.sync_copy(w2_hbm, w2_v)
    pltpu.sync_copy(b2_hbm, b2_v)
    wregs = [w2_v[pl.ds(j * 16, 16)] for j in range(D // 16)]
    b2 = b2_v[...][0]

    def chunk(k, carry):
        base = pl.multiple_of(wid * dpw + k * CH, 8)
        pltpu.sync_copy(u_hbm.at[pl.ds(base, CH)], idx_u)
        pltpu.sync_copy(v_hbm.at[pl.ds(base, CH)], idx_v)
        ca = pltpu.async_copy(a_hbm.at[idx_u], arows, sem)
        cb = pltpu.async_copy(b_hbm.at[idx_v], brows, sem)
        ca.wait()
        cb.wait()
        lanes = lax.iota(jnp.int32, 16)

        def group(g, ecarry):
            res = jnp.zeros((16,), jnp.float32)
            for i in range(16):
                eidx = g * 16 + i
                acc = jnp.zeros((16,), jnp.float32)
                for j in range(D // 16):
                    t = (arows[eidx, pl.ds(j * 16, 16)]
                         + brows[eidx, pl.ds(j * 16, 16)])
                    acc = acc + jnp.maximum(t, 0.0) * wregs[j]
                res = jnp.where(lanes == i, jnp.sum(acc) + b2, res)
            outb[pl.ds(g * 16, 16)] = res
            return ecarry

        lax.fori_loop(0, CH // 16, group, 0)
        pltpu.sync_copy(outb, out_hbm.at[pl.ds(base, CH)])
        return carry

    lax.fori_loop(0, dpw // CH, chunk, 0)


_decoder = pl.kernel(
    _decoder_body,
    out_type=jax.ShapeDtypeStruct((2 * E,), jnp.float32),
    mesh=_mesh,
    scratch_types=[
        pltpu.VMEM((D,), jnp.float32),
        pltpu.VMEM((16,), jnp.float32),
        pltpu.VMEM((CH,), jnp.int32),
        pltpu.VMEM((CH,), jnp.int32),
        pltpu.VMEM((CH, D), jnp.float32),
        pltpu.VMEM((CH, D), jnp.float32),
        pltpu.VMEM((CH,), jnp.float32),
        pltpu.SemaphoreType.DMA,
    ],
    compiler_params=pltpu.CompilerParams(needs_layout_passes=False),
)


def kernel(x, edge_index, pos_edge, neg_edge,
           Wl1, bl1, Wr1, Wl2, bl2, Wr2, Wd1, bd1, Wd2, bd2):
    ei = edge_index.astype(jnp.int32)
    src = ei[0]
    dst = ei[1]
    u_all = jnp.concatenate([pos_edge[0], neg_edge[0]]).astype(jnp.int32)
    v_all = jnp.concatenate([pos_edge[1], neg_edge[1]]).astype(jnp.int32)

    y1, r1 = _enc_prep(x, Wl1, Wr1, bl1.reshape(1, D))
    agg1, cnt = _segsum_cnt(y1, src, dst)
    y2, r2, invb = _combine1(agg1, cnt, r1, Wl2, Wr2, bl2.reshape(1, D))

    agg2 = _segsum(y2, src, dst)
    adec, bdec = _combine2(agg2, r2, invb, Wd1[:, :D], Wd1[:, D:],
                           bd1.reshape(1, D))

    w2 = Wd2.reshape(D)
    b2p = jnp.broadcast_to(bd2.reshape(1), (16,))
    return _decoder(adec, bdec, u_all, v_all, w2, b2p)
